# baseline (speedup 1.0000x reference)
import functools

import jax
import jax.numpy as jnp
from jax import lax
from jax.experimental import pallas as pl
from jax.experimental.pallas import tpu as pltpu
from jax.experimental.pallas import tpu_sc as plsc

N = 50000
E = 1600000
G = 64
F = 100
FP = 128
NC, NS = 2, 16
NW = NC * NS
NB = 64
BS = 784
ROWS = 785
DUMP = 784
WIN = 2000
CBCAP = 4096
FLUSH = 2048
CH = 128
ACH = 32
ANB = 4
ECAP = E + FLUSH + 64
PAD_PACK = DUMP << 16
DEGR = 1568

_mesh = plsc.VectorSubcoreMesh(
    core_axis_name="c", subcore_axis_name="s", num_cores=NC, num_subcores=NS)


def _wid():
    return lax.axis_index("s") * NC + lax.axis_index("c")


def _bucket64(d16):
    return ((d16 >> 4) * 5350) >> 18


@functools.partial(
    pl.kernel,
    out_type=(
        jax.ShapeDtypeStruct((NB * ECAP,), jnp.int32),
        jax.ShapeDtypeStruct((NB * 16,), jnp.int32),
    ),
    mesh=_mesh,
    compiler_params=pltpu.CompilerParams(needs_layout_passes=False),
    scratch_types=[
        pltpu.VMEM((WIN,), jnp.int32),
        pltpu.VMEM((WIN,), jnp.int32),
        pltpu.VMEM((CBCAP,), jnp.int32),
        pltpu.VMEM((CBCAP,), jnp.int32),
        pltpu.VMEM((FLUSH,), jnp.int32),
        pltpu.VMEM((16,), jnp.int32),
    ],
)
def _bin_kernel(src_ref, dst_ref, bins_ref, counts_ref,
                wsrc, wdst, cb0, cb1, padb, stage):
    w = _wid()
    b0 = 2 * w
    rs0 = b0 * BS
    iota16 = lax.iota(jnp.int32, 16)

    def initpad(j, _):
        padb[pl.ds(j * 16, 16)] = jnp.full((16,), PAD_PACK, jnp.int32)
        return 0
    lax.fori_loop(0, FLUSH // 16, initpad, 0)

    def window(win, carry):
        off0, pos0, off1, pos1 = carry
        base = win * WIN
        pltpu.sync_copy(src_ref.at[pl.ds(pl.multiple_of(base, 16), WIN)], wsrc)
        pltpu.sync_copy(dst_ref.at[pl.ds(pl.multiple_of(base, 16), WIN)], wdst)

        def group(g, carry2):
            o0, o1 = carry2
            s16 = wsrc[pl.ds(g * 16, 16)]
            d16 = wdst[pl.ds(g * 16, 16)]
            bk = _bucket64(d16)
            loc = d16 - bk * BS
            pack = s16 | (loc << 16)
            m0 = bk == b0
            m1 = bk == (b0 + 1)
            plsc.store_compressed(cb0.at[pl.ds(o0, 16)], pack, mask=m0)
            plsc.store_compressed(cb1.at[pl.ds(o1, 16)], pack, mask=m1)
            o0 = o0 + jnp.max(plsc.all_reduce_population_count(m0))
            o1 = o1 + jnp.max(plsc.all_reduce_population_count(m1))
            return o0, o1

        off0, off1 = lax.fori_loop(0, WIN // 16, group, (off0, off1))

        def flush(cb, bkt, off, pos):
            @pl.when(off >= FLUSH)
            def _():
                pltpu.sync_copy(cb.at[pl.ds(0, FLUSH)],
                                bins_ref.at[pl.ds(pl.multiple_of(bkt * ECAP + pos, 16), FLUSH)])

                def shift(j, _):
                    v = cb[pl.ds(FLUSH + j * 16, 16)]
                    cb[pl.ds(j * 16, 16)] = v
                    return 0
                lax.fori_loop(0, (CBCAP - FLUSH) // 16, shift, 0)
            did = (off >= FLUSH).astype(jnp.int32)
            return off - did * FLUSH, pos + did * FLUSH

        off0, pos0 = flush(cb0, b0, off0, pos0)
        off1, pos1 = flush(cb1, b0 + 1, off1, pos1)
        return off0, pos0, off1, pos1

    off0, pos0, off1, pos1 = lax.fori_loop(
        0, E // WIN, window, (jnp.int32(0),) * 4)

    def final(cb, bkt, off, pos):
        k = (16 - (off & 15)) & 15
        plsc.store_compressed(cb.at[pl.ds(off, 16)],
                              jnp.full((16,), PAD_PACK, jnp.int32),
                              mask=iota16 < k)
        off = off + k
        pltpu.sync_copy(cb.at[pl.ds(0, FLUSH)],
                        bins_ref.at[pl.ds(pl.multiple_of(bkt * ECAP + pos, 16), FLUSH)])
        total = pos + off
        pltpu.sync_copy(padb, bins_ref.at[pl.ds(pl.multiple_of(bkt * ECAP + total, 16), FLUSH)])
        stage[pl.ds(0, 16)] = jnp.zeros((16,), jnp.int32) + total
        pltpu.sync_copy(stage, counts_ref.at[pl.ds(pl.multiple_of(bkt * 16, 16), 16)])

    final(cb0, b0, off0, pos0)
    final(cb1, b0 + 1, off1, pos1)


@functools.partial(
    pl.kernel,
    out_type=jax.ShapeDtypeStruct((N * 16,), jnp.float32),
    mesh=_mesh,
    compiler_params=pltpu.CompilerParams(needs_layout_passes=False),
    scratch_types=[
        pltpu.VMEM((ROWS * 16,), jnp.float32),
        pltpu.VMEM((CH,), jnp.int32),
        pltpu.VMEM((16,), jnp.int32),
    ],
)
def _deg_kernel(bins_ref, counts_ref, deg_ref, dega, packv, cntv):
    w = _wid()
    zeros16f = jnp.zeros((16,), jnp.float32)
    onehot = jnp.where(lax.iota(jnp.int32, 16) == 0, 1.0, 0.0)

    for sb in range(2):
        b = 2 * w + sb
        rs0 = b * BS

        def zrow(r, _):
            dega[pl.ds(r * 16, 16)] = zeros16f
            return 0
        lax.fori_loop(0, ROWS, zrow, 0)

        pltpu.sync_copy(counts_ref.at[pl.ds(pl.multiple_of(b * 16, 16), 16)],
                        cntv)
        n = jnp.max(cntv[...])
        nch = (n + CH - 1) >> 7

        def chunk(c, _):
            pos = c * CH
            pltpu.sync_copy(
                bins_ref.at[pl.ds(pl.multiple_of(b * ECAP + pos, 16), CH)],
                packv)

            def grp(g, _):
                loc16 = packv[pl.ds(g * 16, 16)] >> 16
                for j in range(16):
                    loc = loc16[j]
                    plsc.addupdate(dega.at[pl.ds(loc * 16, 16)], onehot)
                return 0
            lax.fori_loop(0, CH // 16, grp, 0)
            return 0

        lax.fori_loop(0, nch, chunk, 0)

        pltpu.sync_copy(dega.at[pl.ds(0, 608 * 16)],
                        deg_ref.at[pl.ds(pl.multiple_of(rs0 * 16, 16),
                                         608 * 16)])

        @pl.when(b < NB - 1)
        def _():
            pltpu.sync_copy(
                dega.at[pl.ds(608 * 16, 176 * 16)],
                deg_ref.at[pl.ds(pl.multiple_of((rs0 + 608) * 16, 16),
                                 176 * 16)])


@functools.partial(
    pl.kernel,
    out_type=jax.ShapeDtypeStruct((N * FP,), jnp.float32),
    mesh=_mesh,
    compiler_params=pltpu.CompilerParams(needs_layout_passes=False),
    scratch_types=[
        pltpu.VMEM((ROWS * FP,), jnp.float32),
        pltpu.VMEM((ANB * ACH,), jnp.int32),
        [pltpu.VMEM((ACH,), jnp.int32) for _ in range(ANB)],
        [pltpu.VMEM((ACH,), jnp.int32) for _ in range(ANB)],
        [pltpu.VMEM((ACH, FP), jnp.float32) for _ in range(ANB)],
        pltpu.VMEM((16,), jnp.int32),
        [pltpu.SemaphoreType.DMA for _ in range(ANB)],
        pltpu.SemaphoreType.DMA,
    ],
)
def _acc_kernel(p_ref, bins_ref, counts_ref, out_ref,
                acc, packq, srcs, locs, rows, cntv, semr, semp):
    w = _wid()
    zeros16f = jnp.zeros((16,), jnp.float32)

    for sb in range(2):
        b = 2 * w + sb
        rs0 = b * BS
        base = b * ECAP

        def zrow(r, _):
            acc[pl.ds(r * 16, 16)] = zeros16f
            return 0
        lax.fori_loop(0, ROWS * FP // 16, zrow, 0)

        pltpu.sync_copy(counts_ref.at[pl.ds(pl.multiple_of(b * 16, 16), 16)], cntv)
        n = jnp.max(cntv[...])
        nq = (n + ANB * ACH - 1) >> 7

        def build_issue(qi, bf):
            for g in range(ACH // 16):
                pk = packq[pl.ds(bf * ACH + g * 16, 16)]
                srcs[bf][pl.ds(g * 16, 16)] = pk & 0xFFFF
                locs[bf][pl.ds(g * 16, 16)] = pk >> 16
            pltpu.async_copy(p_ref.at[srcs[bf]], rows[bf], semr[bf])

        def load_packq(qi):
            pltpu.async_copy(
                bins_ref.at[pl.ds(pl.multiple_of(base + qi * (ANB * ACH), 16),
                                  ANB * ACH)],
                packq, semp)

        iota16 = lax.iota(jnp.int32, 16)

        def accum(bf):
            pltpu.make_async_copy(p_ref.at[srcs[bf]], rows[bf],
                                  semr[bf]).wait()

            def grp(g, _):
                abase = locs[bf][pl.ds(g * 16, 16)] * FP
                e16 = g * 16 + iota16
                cvec = jnp.zeros((16,), jnp.int32)
                for c in range(FP):
                    v = plsc.load_gather(rows[bf], [e16, cvec])
                    plsc.addupdate_scatter(acc, [abase + c], v)
                    cvec = cvec + 1
                return 0
            lax.fori_loop(0, ACH // 16, grp, 0)

        @pl.when(n > 0)
        def _():
            pltpu.async_copy(
                bins_ref.at[pl.ds(pl.multiple_of(base, 16), ANB * ACH)],
                packq, semp)
            pltpu.make_async_copy(
                bins_ref.at[pl.ds(pl.multiple_of(base, 16), ANB * ACH)],
                packq, semp).wait()
            for bf in range(ANB):
                build_issue(0, bf)
            load_packq(1)

            def quad(qi, _):
                pltpu.make_async_copy(
                    bins_ref.at[pl.ds(pl.multiple_of(base, 16), ANB * ACH)],
                    packq, semp).wait()
                for bf in range(ANB):
                    accum(bf)
                    build_issue(qi, bf)
                load_packq(qi + 1)
                return 0
            lax.fori_loop(1, nq, quad, 0)

            pltpu.make_async_copy(
                bins_ref.at[pl.ds(pl.multiple_of(base, 16), ANB * ACH)],
                packq, semp).wait()
            for bf in range(ANB):
                accum(bf)

        pltpu.sync_copy(acc.at[pl.ds(0, 608 * FP)],
                        out_ref.at[pl.ds(pl.multiple_of(rs0 * FP, 16),
                                         608 * FP)])

        @pl.when(b < NB - 1)
        def _():
            pltpu.sync_copy(
                acc.at[pl.ds(608 * FP, 176 * FP)],
                out_ref.at[pl.ds(pl.multiple_of((rs0 + 608) * FP, 16),
                                 176 * FP)])


BK = 512
NBK = 98


def _h0p1_body(x_ref, w0_ref, b0_ref, w1_ref, deg_ref, o_ref):
    h0 = jnp.maximum(
        jnp.dot(x_ref[...], w0_ref[...], preferred_element_type=jnp.float32)
        + b0_ref[...], 0.0)
    dv = lax.rsqrt(deg_ref[:, 0:1] + 1.0)
    o_ref[...] = jnp.dot(h0, w1_ref[...],
                         preferred_element_type=jnp.float32) * dv


def _step_body(agg_ref, p_ref, deg_ref, b_ref, w_ref, o_ref):
    dv = lax.rsqrt(deg_ref[:, 0:1] + 1.0)
    t = (agg_ref[...] + p_ref[...]) * dv
    h = jnp.maximum(t[:, :F] + b_ref[...], 0.0)
    o_ref[...] = jnp.dot(h, w_ref[...],
                         preferred_element_type=jnp.float32) * dv


def _final_body(agg_ref, p_ref, deg_ref, b_ref, batch_ref, g_ref):
    pid = pl.program_id(0)

    @pl.when(pid == 0)
    def _():
        g_ref[...] = jnp.full((G, F), -jnp.inf, jnp.float32)

    dv = lax.rsqrt(deg_ref[:, 0:1] + 1.0)
    t = (agg_ref[...] + p_ref[...]) * dv
    h = jnp.maximum(t[:, :F] + b_ref[...], 0.0)
    rows = pid * BK + lax.broadcasted_iota(jnp.int32, (BK, 1), 0)
    h = jnp.where(rows < N, h, -jnp.inf)
    bt = batch_ref[:, 0:1]
    lo = jnp.min(bt)
    hi = jnp.max(bt)

    def body(gid, _):
        m = bt == gid
        colmax = jnp.max(jnp.where(m, h, -jnp.inf), axis=0)
        cur = g_ref[pl.ds(gid, 1), :]
        g_ref[pl.ds(gid, 1), :] = jnp.maximum(cur, colmax[None, :])
        return 0

    lax.fori_loop(lo, hi + 1, body, 0)


def _head_body(g_ref, w4_ref, b4_ref, w5_ref, b5_ref, w6_ref, b6_ref, o_ref):
    a = jnp.maximum(jnp.dot(g_ref[...], w4_ref[...],
                            preferred_element_type=jnp.float32)
                    + b4_ref[...], 0.0)
    a = jnp.maximum(jnp.dot(a, w5_ref[...],
                            preferred_element_type=jnp.float32)
                    + b5_ref[...], 0.0)
    z = jnp.dot(a, w6_ref[...], preferred_element_type=jnp.float32) \
        + b6_ref[...]
    z = z - jnp.max(z, axis=0, keepdims=True)
    ez = jnp.exp(z)
    o_ref[...] = ez / jnp.sum(ez, axis=0, keepdims=True)


def _rows_spec(width):
    return pl.BlockSpec((BK, width), lambda i: (i, 0))


def _full_spec(shape):
    return pl.BlockSpec(shape, lambda *a: tuple(0 for _ in shape))


def _tc_h0p1(x, w0, b0r, w1p, deg16):
    return pl.pallas_call(
        _h0p1_body,
        grid=(NBK,),
        in_specs=[_rows_spec(19), _full_spec((19, F)), _full_spec((1, F)),
                  _full_spec((F, FP)), _rows_spec(16)],
        out_specs=_rows_spec(FP),
        out_shape=jax.ShapeDtypeStruct((N, FP), jnp.float32),
    )(x, w0, b0r, w1p, deg16)


def _tc_step(agg, p, deg16, br, wp):
    return pl.pallas_call(
        _step_body,
        grid=(NBK,),
        in_specs=[_rows_spec(FP), _rows_spec(FP), _rows_spec(16),
                  _full_spec((1, F)), _full_spec((F, FP))],
        out_specs=_rows_spec(FP),
        out_shape=jax.ShapeDtypeStruct((N, FP), jnp.float32),
    )(agg, p, deg16, br, wp)


def _tc_final(agg, p, deg16, br, batch16):
    return pl.pallas_call(
        _final_body,
        grid=(NBK,),
        in_specs=[_rows_spec(FP), _rows_spec(FP), _rows_spec(16),
                  _full_spec((1, F)), _rows_spec(16)],
        out_specs=_full_spec((G, F)),
        out_shape=jax.ShapeDtypeStruct((G, F), jnp.float32),
    )(agg, p, deg16, br, batch16)


def _tc_head(g, w4, b4r, w5, b5r, w6, b6r):
    return pl.pallas_call(
        _head_body,
        in_specs=[_full_spec((G, F)), _full_spec((F, 80)),
                  _full_spec((1, 80)), _full_spec((80, 60)),
                  _full_spec((1, 60)), _full_spec((60, 6)),
                  _full_spec((1, 6))],
        out_specs=_full_spec((G, 6)),
        out_shape=jax.ShapeDtypeStruct((G, 6), jnp.float32),
    )(g, w4, b4r, w5, b5r, w6, b6r)


def kernel(x, edge_index, batch, W0, b0, W1, b1, W2, b2, W3, b3,
           W4, b4, W5, b5, W6, b6):
    padW = lambda w: jnp.pad(w, ((0, 0), (0, FP - F)))
    w1p, w2p, w3p = padW(W1), padW(W2), padW(W3)
    b0r, b1r, b2r, b3r = (b.reshape(1, F) for b in (b0, b1, b2, b3))
    b4r, b5r, b6r = b4.reshape(1, 80), b5.reshape(1, 60), b6.reshape(1, 6)
    batchp = jnp.concatenate(
        [batch, jnp.full((NBK * BK - N,), G - 1, jnp.int32)])
    batch16 = jnp.broadcast_to(batchp[:, None], (NBK * BK, 16))

    bins, counts = _bin_kernel(edge_index[0], edge_index[1])
    degflat = _deg_kernel(bins, counts)
    deg16 = degflat.reshape(N, 16)

    p1 = _tc_h0p1(x, W0, b0r, w1p, deg16)
    agg1 = _acc_kernel(p1, bins, counts).reshape(N, FP)
    p2 = _tc_step(agg1, p1, deg16, b1r, w2p)
    agg2 = _acc_kernel(p2, bins, counts).reshape(N, FP)
    p3 = _tc_step(agg2, p2, deg16, b2r, w3p)
    agg3 = _acc_kernel(p3, bins, counts).reshape(N, FP)
    g = _tc_final(agg3, p3, deg16, b3r, batch16)
    return _tc_head(g, W4, b4r, W5, b5r, W6, b6r)


# speedup vs baseline: 5.0612x; 5.0612x over previous
import functools

import jax
import jax.numpy as jnp
from jax import lax
from jax.experimental import pallas as pl
from jax.experimental.pallas import tpu as pltpu
from jax.experimental.pallas import tpu_sc as plsc

N = 50000
E = 1600000
G = 64
F = 100
FP = 128
NC, NS = 2, 16
NW = NC * NS
NB = 64
BS = 784
ROWS = 785
DUMP = 784
WIN = 2000
CBCAP = 4096
FLUSH = 2048
CH = 128
ACH = 32
ANB = 4
ECAP = E + FLUSH + 64
PAD_PACK = DUMP << 16
DEGR = 1568

_mesh = plsc.VectorSubcoreMesh(
    core_axis_name="c", subcore_axis_name="s", num_cores=NC, num_subcores=NS)


def _wid():
    return lax.axis_index("s") * NC + lax.axis_index("c")


def _bucket64(d16):
    return ((d16 >> 4) * 5350) >> 18


@functools.partial(
    pl.kernel,
    out_type=(
        jax.ShapeDtypeStruct((NB * ECAP,), jnp.int32),
        jax.ShapeDtypeStruct((NB * 16,), jnp.int32),
    ),
    mesh=_mesh,
    compiler_params=pltpu.CompilerParams(needs_layout_passes=False),
    scratch_types=[
        pltpu.VMEM((WIN,), jnp.int32),
        pltpu.VMEM((WIN,), jnp.int32),
        pltpu.VMEM((CBCAP,), jnp.int32),
        pltpu.VMEM((CBCAP,), jnp.int32),
        pltpu.VMEM((FLUSH,), jnp.int32),
        pltpu.VMEM((16,), jnp.int32),
    ],
)
def _bin_kernel(src_ref, dst_ref, bins_ref, counts_ref,
                wsrc, wdst, cb0, cb1, padb, stage):
    w = _wid()
    b0 = 2 * w
    rs0 = b0 * BS
    iota16 = lax.iota(jnp.int32, 16)

    def initpad(j, _):
        padb[pl.ds(j * 16, 16)] = jnp.full((16,), PAD_PACK, jnp.int32)
        return 0
    lax.fori_loop(0, FLUSH // 16, initpad, 0)

    def window(win, carry):
        off0, pos0, off1, pos1 = carry
        base = win * WIN
        pltpu.sync_copy(src_ref.at[pl.ds(pl.multiple_of(base, 16), WIN)], wsrc)
        pltpu.sync_copy(dst_ref.at[pl.ds(pl.multiple_of(base, 16), WIN)], wdst)

        def group(g, carry2):
            o0, o1 = carry2
            s16 = wsrc[pl.ds(g * 16, 16)]
            d16 = wdst[pl.ds(g * 16, 16)]
            bk = _bucket64(d16)
            loc = d16 - bk * BS
            pack = s16 | (loc << 16)
            m0 = bk == b0
            m1 = bk == (b0 + 1)
            plsc.store_compressed(cb0.at[pl.ds(o0, 16)], pack, mask=m0)
            plsc.store_compressed(cb1.at[pl.ds(o1, 16)], pack, mask=m1)
            o0 = o0 + jnp.max(plsc.all_reduce_population_count(m0))
            o1 = o1 + jnp.max(plsc.all_reduce_population_count(m1))
            return o0, o1

        off0, off1 = lax.fori_loop(0, WIN // 16, group, (off0, off1))

        def flush(cb, bkt, off, pos):
            @pl.when(off >= FLUSH)
            def _():
                pltpu.sync_copy(cb.at[pl.ds(0, FLUSH)],
                                bins_ref.at[pl.ds(pl.multiple_of(bkt * ECAP + pos, 16), FLUSH)])

                def shift(j, _):
                    v = cb[pl.ds(FLUSH + j * 16, 16)]
                    cb[pl.ds(j * 16, 16)] = v
                    return 0
                lax.fori_loop(0, (CBCAP - FLUSH) // 16, shift, 0)
            did = (off >= FLUSH).astype(jnp.int32)
            return off - did * FLUSH, pos + did * FLUSH

        off0, pos0 = flush(cb0, b0, off0, pos0)
        off1, pos1 = flush(cb1, b0 + 1, off1, pos1)
        return off0, pos0, off1, pos1

    off0, pos0, off1, pos1 = lax.fori_loop(
        0, E // WIN, window, (jnp.int32(0),) * 4)

    def final(cb, bkt, off, pos):
        k = (16 - (off & 15)) & 15
        plsc.store_compressed(cb.at[pl.ds(off, 16)],
                              jnp.full((16,), PAD_PACK, jnp.int32),
                              mask=iota16 < k)
        off = off + k
        pltpu.sync_copy(cb.at[pl.ds(0, FLUSH)],
                        bins_ref.at[pl.ds(pl.multiple_of(bkt * ECAP + pos, 16), FLUSH)])
        total = pos + off
        pltpu.sync_copy(padb, bins_ref.at[pl.ds(pl.multiple_of(bkt * ECAP + total, 16), FLUSH)])
        stage[pl.ds(0, 16)] = jnp.zeros((16,), jnp.int32) + total
        pltpu.sync_copy(stage, counts_ref.at[pl.ds(pl.multiple_of(bkt * 16, 16), 16)])

    final(cb0, b0, off0, pos0)
    final(cb1, b0 + 1, off1, pos1)


@functools.partial(
    pl.kernel,
    out_type=jax.ShapeDtypeStruct((N * 16,), jnp.float32),
    mesh=_mesh,
    compiler_params=pltpu.CompilerParams(needs_layout_passes=False),
    scratch_types=[
        pltpu.VMEM((ROWS * 16,), jnp.float32),
        pltpu.VMEM((CH,), jnp.int32),
        pltpu.VMEM((16,), jnp.int32),
    ],
)
def _deg_kernel(bins_ref, counts_ref, deg_ref, dega, packv, cntv):
    w = _wid()
    zeros16f = jnp.zeros((16,), jnp.float32)
    onehot = jnp.where(lax.iota(jnp.int32, 16) == 0, 1.0, 0.0)

    for sb in range(2):
        b = 2 * w + sb
        rs0 = b * BS

        def zrow(r, _):
            dega[pl.ds(r * 16, 16)] = zeros16f
            return 0
        lax.fori_loop(0, ROWS, zrow, 0)

        pltpu.sync_copy(counts_ref.at[pl.ds(pl.multiple_of(b * 16, 16), 16)],
                        cntv)
        n = jnp.max(cntv[...])
        nch = (n + CH - 1) >> 7

        def chunk(c, _):
            pos = c * CH
            pltpu.sync_copy(
                bins_ref.at[pl.ds(pl.multiple_of(b * ECAP + pos, 16), CH)],
                packv)

            def grp(g, _):
                loc16 = packv[pl.ds(g * 16, 16)] >> 16
                for j in range(16):
                    loc = loc16[j]
                    plsc.addupdate(dega.at[pl.ds(loc * 16, 16)], onehot)
                return 0
            lax.fori_loop(0, CH // 16, grp, 0)
            return 0

        lax.fori_loop(0, nch, chunk, 0)

        pltpu.sync_copy(dega.at[pl.ds(0, 608 * 16)],
                        deg_ref.at[pl.ds(pl.multiple_of(rs0 * 16, 16),
                                         608 * 16)])

        @pl.when(b < NB - 1)
        def _():
            pltpu.sync_copy(
                dega.at[pl.ds(608 * 16, 176 * 16)],
                deg_ref.at[pl.ds(pl.multiple_of((rs0 + 608) * 16, 16),
                                 176 * 16)])


@functools.partial(
    pl.kernel,
    out_type=jax.ShapeDtypeStruct((N * FP,), jnp.float32),
    mesh=_mesh,
    compiler_params=pltpu.CompilerParams(needs_layout_passes=False),
    scratch_types=[
        pltpu.VMEM((ROWS * FP,), jnp.float32),
        pltpu.VMEM((ANB * ACH,), jnp.int32),
        [pltpu.VMEM((ACH,), jnp.int32) for _ in range(ANB)],
        [pltpu.VMEM((ACH,), jnp.int32) for _ in range(ANB)],
        [pltpu.VMEM((ACH, FP), jnp.float32) for _ in range(ANB)],
        pltpu.VMEM((16,), jnp.int32),
        [pltpu.SemaphoreType.DMA for _ in range(ANB)],
        pltpu.SemaphoreType.DMA,
    ],
)
def _acc_kernel(p_ref, bins_ref, counts_ref, out_ref,
                acc, packq, srcs, locs, rows, cntv, semr, semp):
    w = _wid()
    zeros16f = jnp.zeros((16,), jnp.float32)

    for sb in range(2):
        b = 2 * w + sb
        rs0 = b * BS
        base = b * ECAP

        def zrow(r, _):
            acc[pl.ds(r * 16, 16)] = zeros16f
            return 0
        lax.fori_loop(0, ROWS * FP // 16, zrow, 0)

        pltpu.sync_copy(counts_ref.at[pl.ds(pl.multiple_of(b * 16, 16), 16)], cntv)
        n = jnp.max(cntv[...])
        nq = (n + ANB * ACH - 1) >> 7

        def build_issue(qi, bf):
            for g in range(ACH // 16):
                pk = packq[pl.ds(bf * ACH + g * 16, 16)]
                srcs[bf][pl.ds(g * 16, 16)] = pk & 0xFFFF
                locs[bf][pl.ds(g * 16, 16)] = pk >> 16
            pltpu.async_copy(p_ref.at[srcs[bf]], rows[bf], semr[bf])

        def load_packq(qi):
            pltpu.async_copy(
                bins_ref.at[pl.ds(pl.multiple_of(base + qi * (ANB * ACH), 16),
                                  ANB * ACH)],
                packq, semp)

        iota16 = lax.iota(jnp.int32, 16)

        def accum(bf):
            pltpu.make_async_copy(p_ref.at[srcs[bf]], rows[bf],
                                  semr[bf]).wait()

            def grp(g, _):
                loc16 = locs[bf][pl.ds(g * 16, 16)]
                for j in range(16):
                    loc = loc16[j]
                    e = g * 16 + j
                    base = loc * FP
                    vs = [rows[bf][e, pl.ds(f * 16, 16)]
                          for f in range(FP // 16)]
                    for f in range(FP // 16):
                        plsc.addupdate(acc.at[pl.ds(base + f * 16, 16)],
                                       vs[f])
                return 0
            lax.fori_loop(0, ACH // 16, grp, 0)

        @pl.when(n > 0)
        def _():
            pltpu.async_copy(
                bins_ref.at[pl.ds(pl.multiple_of(base, 16), ANB * ACH)],
                packq, semp)
            pltpu.make_async_copy(
                bins_ref.at[pl.ds(pl.multiple_of(base, 16), ANB * ACH)],
                packq, semp).wait()
            for bf in range(ANB):
                build_issue(0, bf)
            load_packq(1)

            def quad(qi, _):
                pltpu.make_async_copy(
                    bins_ref.at[pl.ds(pl.multiple_of(base, 16), ANB * ACH)],
                    packq, semp).wait()
                for bf in range(ANB):
                    accum(bf)
                    build_issue(qi, bf)
                load_packq(qi + 1)
                return 0
            lax.fori_loop(1, nq, quad, 0)

            pltpu.make_async_copy(
                bins_ref.at[pl.ds(pl.multiple_of(base, 16), ANB * ACH)],
                packq, semp).wait()
            for bf in range(ANB):
                accum(bf)

        pltpu.sync_copy(acc.at[pl.ds(0, 608 * FP)],
                        out_ref.at[pl.ds(pl.multiple_of(rs0 * FP, 16),
                                         608 * FP)])

        @pl.when(b < NB - 1)
        def _():
            pltpu.sync_copy(
                acc.at[pl.ds(608 * FP, 176 * FP)],
                out_ref.at[pl.ds(pl.multiple_of((rs0 + 608) * FP, 16),
                                 176 * FP)])


BK = 512
NBK = 98


def _h0p1_body(x_ref, w0_ref, b0_ref, w1_ref, deg_ref, o_ref):
    h0 = jnp.maximum(
        jnp.dot(x_ref[...], w0_ref[...], preferred_element_type=jnp.float32)
        + b0_ref[...], 0.0)
    dv = lax.rsqrt(deg_ref[:, 0:1] + 1.0)
    o_ref[...] = jnp.dot(h0, w1_ref[...],
                         preferred_element_type=jnp.float32) * dv


def _step_body(agg_ref, p_ref, deg_ref, b_ref, w_ref, o_ref):
    dv = lax.rsqrt(deg_ref[:, 0:1] + 1.0)
    t = (agg_ref[...] + p_ref[...]) * dv
    h = jnp.maximum(t[:, :F] + b_ref[...], 0.0)
    o_ref[...] = jnp.dot(h, w_ref[...],
                         preferred_element_type=jnp.float32) * dv


def _final_body(agg_ref, p_ref, deg_ref, b_ref, batch_ref, g_ref):
    pid = pl.program_id(0)

    @pl.when(pid == 0)
    def _():
        g_ref[...] = jnp.full((G, F), -jnp.inf, jnp.float32)

    dv = lax.rsqrt(deg_ref[:, 0:1] + 1.0)
    t = (agg_ref[...] + p_ref[...]) * dv
    h = jnp.maximum(t[:, :F] + b_ref[...], 0.0)
    rows = pid * BK + lax.broadcasted_iota(jnp.int32, (BK, 1), 0)
    h = jnp.where(rows < N, h, -jnp.inf)
    bt = batch_ref[:, 0:1]
    lo = jnp.min(bt)
    hi = jnp.max(bt)

    def body(gid, _):
        m = bt == gid
        colmax = jnp.max(jnp.where(m, h, -jnp.inf), axis=0)
        cur = g_ref[pl.ds(gid, 1), :]
        g_ref[pl.ds(gid, 1), :] = jnp.maximum(cur, colmax[None, :])
        return 0

    lax.fori_loop(lo, hi + 1, body, 0)


def _head_body(g_ref, w4_ref, b4_ref, w5_ref, b5_ref, w6_ref, b6_ref, o_ref):
    a = jnp.maximum(jnp.dot(g_ref[...], w4_ref[...],
                            preferred_element_type=jnp.float32)
                    + b4_ref[...], 0.0)
    a = jnp.maximum(jnp.dot(a, w5_ref[...],
                            preferred_element_type=jnp.float32)
                    + b5_ref[...], 0.0)
    z = jnp.dot(a, w6_ref[...], preferred_element_type=jnp.float32) \
        + b6_ref[...]
    z = z - jnp.max(z, axis=0, keepdims=True)
    ez = jnp.exp(z)
    o_ref[...] = ez / jnp.sum(ez, axis=0, keepdims=True)


def _rows_spec(width):
    return pl.BlockSpec((BK, width), lambda i: (i, 0))


def _full_spec(shape):
    return pl.BlockSpec(shape, lambda *a: tuple(0 for _ in shape))


def _tc_h0p1(x, w0, b0r, w1p, deg16):
    return pl.pallas_call(
        _h0p1_body,
        grid=(NBK,),
        in_specs=[_rows_spec(19), _full_spec((19, F)), _full_spec((1, F)),
                  _full_spec((F, FP)), _rows_spec(16)],
        out_specs=_rows_spec(FP),
        out_shape=jax.ShapeDtypeStruct((N, FP), jnp.float32),
    )(x, w0, b0r, w1p, deg16)


def _tc_step(agg, p, deg16, br, wp):
    return pl.pallas_call(
        _step_body,
        grid=(NBK,),
        in_specs=[_rows_spec(FP), _rows_spec(FP), _rows_spec(16),
                  _full_spec((1, F)), _full_spec((F, FP))],
        out_specs=_rows_spec(FP),
        out_shape=jax.ShapeDtypeStruct((N, FP), jnp.float32),
    )(agg, p, deg16, br, wp)


def _tc_final(agg, p, deg16, br, batch16):
    return pl.pallas_call(
        _final_body,
        grid=(NBK,),
        in_specs=[_rows_spec(FP), _rows_spec(FP), _rows_spec(16),
                  _full_spec((1, F)), _rows_spec(16)],
        out_specs=_full_spec((G, F)),
        out_shape=jax.ShapeDtypeStruct((G, F), jnp.float32),
    )(agg, p, deg16, br, batch16)


def _tc_head(g, w4, b4r, w5, b5r, w6, b6r):
    return pl.pallas_call(
        _head_body,
        in_specs=[_full_spec((G, F)), _full_spec((F, 80)),
                  _full_spec((1, 80)), _full_spec((80, 60)),
                  _full_spec((1, 60)), _full_spec((60, 6)),
                  _full_spec((1, 6))],
        out_specs=_full_spec((G, 6)),
        out_shape=jax.ShapeDtypeStruct((G, 6), jnp.float32),
    )(g, w4, b4r, w5, b5r, w6, b6r)


def kernel(x, edge_index, batch, W0, b0, W1, b1, W2, b2, W3, b3,
           W4, b4, W5, b5, W6, b6):
    padW = lambda w: jnp.pad(w, ((0, 0), (0, FP - F)))
    w1p, w2p, w3p = padW(W1), padW(W2), padW(W3)
    b0r, b1r, b2r, b3r = (b.reshape(1, F) for b in (b0, b1, b2, b3))
    b4r, b5r, b6r = b4.reshape(1, 80), b5.reshape(1, 60), b6.reshape(1, 6)
    batchp = jnp.concatenate(
        [batch, jnp.full((NBK * BK - N,), G - 1, jnp.int32)])
    batch16 = jnp.broadcast_to(batchp[:, None], (NBK * BK, 16))

    bins, counts = _bin_kernel(edge_index[0], edge_index[1])
    degflat = _deg_kernel(bins, counts)
    deg16 = degflat.reshape(N, 16)

    p1 = _tc_h0p1(x, W0, b0r, w1p, deg16)
    agg1 = _acc_kernel(p1, bins, counts).reshape(N, FP)
    p2 = _tc_step(agg1, p1, deg16, b1r, w2p)
    agg2 = _acc_kernel(p2, bins, counts).reshape(N, FP)
    p3 = _tc_step(agg2, p2, deg16, b2r, w3p)
    agg3 = _acc_kernel(p3, bins, counts).reshape(N, FP)
    g = _tc_final(agg3, p3, deg16, b3r, batch16)
    return _tc_head(g, W4, b4r, W5, b5r, W6, b6r)


# speedup vs baseline: 6.4967x; 1.2836x over previous
import functools

import jax
import jax.numpy as jnp
from jax import lax
from jax.experimental import pallas as pl
from jax.experimental.pallas import tpu as pltpu
from jax.experimental.pallas import tpu_sc as plsc

N = 50000
E = 1600000
G = 64
F = 100
FP = 128
NC, NS = 2, 16
NW = NC * NS
NB = 64
BS = 784
ROWS = 785
DUMP = 784
WIN = 2000
CBCAP = 4096
FLUSH = 2048
CH = 128
ACH = 32
ANB = 4
ECAP = E + FLUSH + 64
PAD_PACK = DUMP << 16
DEGR = 1568

_mesh = plsc.VectorSubcoreMesh(
    core_axis_name="c", subcore_axis_name="s", num_cores=NC, num_subcores=NS)


def _wid():
    return lax.axis_index("s") * NC + lax.axis_index("c")


def _bucket64(d16):
    return ((d16 >> 4) * 5350) >> 18


@functools.partial(
    pl.kernel,
    out_type=(
        jax.ShapeDtypeStruct((NB * ECAP,), jnp.int32),
        jax.ShapeDtypeStruct((NB * 16,), jnp.int32),
    ),
    mesh=_mesh,
    compiler_params=pltpu.CompilerParams(needs_layout_passes=False),
    scratch_types=[
        [pltpu.VMEM((WIN,), jnp.int32) for _ in range(2)],
        [pltpu.VMEM((WIN,), jnp.int32) for _ in range(2)],
        pltpu.VMEM((CBCAP,), jnp.int32),
        pltpu.VMEM((CBCAP,), jnp.int32),
        pltpu.VMEM((FLUSH,), jnp.int32),
        pltpu.VMEM((16,), jnp.int32),
        [pltpu.SemaphoreType.DMA for _ in range(2)],
    ],
)
def _bin_kernel(src_ref, dst_ref, bins_ref, counts_ref,
                wsrc, wdst, cb0, cb1, padb, stage, semw):
    w = _wid()
    b0 = 2 * w
    rs0 = b0 * BS
    iota16 = lax.iota(jnp.int32, 16)

    def initpad(j, _):
        padb[pl.ds(j * 16, 16)] = jnp.full((16,), PAD_PACK, jnp.int32)
        return 0
    lax.fori_loop(0, FLUSH // 16, initpad, 0)

    def issue_win(win, bf):
        base = pl.multiple_of(win * WIN, 16)
        pltpu.async_copy(src_ref.at[pl.ds(base, WIN)], wsrc[bf], semw[bf])
        pltpu.async_copy(dst_ref.at[pl.ds(base, WIN)], wdst[bf], semw[bf])

    def wait_win(win, bf):
        base = pl.multiple_of(win * WIN, 16)
        pltpu.make_async_copy(src_ref.at[pl.ds(base, WIN)], wsrc[bf],
                              semw[bf]).wait()
        pltpu.make_async_copy(dst_ref.at[pl.ds(base, WIN)], wdst[bf],
                              semw[bf]).wait()

    def window(win, bf, carry):
        off0, pos0, off1, pos1 = carry
        wait_win(win, bf)

        def group(g, carry2):
            o0, o1 = carry2
            s16 = wsrc[bf][pl.ds(g * 16, 16)]
            d16 = wdst[bf][pl.ds(g * 16, 16)]
            bk = _bucket64(d16)
            loc = d16 - bk * BS
            pack = s16 | (loc << 16)
            m0 = bk == b0
            m1 = bk == (b0 + 1)
            plsc.store_compressed(cb0.at[pl.ds(o0, 16)], pack, mask=m0)
            plsc.store_compressed(cb1.at[pl.ds(o1, 16)], pack, mask=m1)
            o0 = o0 + plsc.all_reduce_population_count(m0)[0]
            o1 = o1 + plsc.all_reduce_population_count(m1)[0]
            return o0, o1

        off0, off1 = lax.fori_loop(0, WIN // 16, group, (off0, off1))

        @pl.when(win + 2 < E // WIN)
        def _():
            issue_win(win + 2, bf)

        def flush(cb, bkt, off, pos):
            @pl.when(off >= FLUSH)
            def _():
                pltpu.sync_copy(cb.at[pl.ds(0, FLUSH)],
                                bins_ref.at[pl.ds(pl.multiple_of(bkt * ECAP + pos, 16), FLUSH)])

                def shift(j, _):
                    v = cb[pl.ds(FLUSH + j * 16, 16)]
                    cb[pl.ds(j * 16, 16)] = v
                    return 0
                lax.fori_loop(0, (CBCAP - FLUSH) // 16, shift, 0)
            did = (off >= FLUSH).astype(jnp.int32)
            return off - did * FLUSH, pos + did * FLUSH

        off0, pos0 = flush(cb0, b0, off0, pos0)
        off1, pos1 = flush(cb1, b0 + 1, off1, pos1)
        return off0, pos0, off1, pos1

    issue_win(0, 0)
    issue_win(1, 1)

    def winpair(pi, carry):
        carry = window(2 * pi, 0, carry)
        carry = window(2 * pi + 1, 1, carry)
        return carry

    off0, pos0, off1, pos1 = lax.fori_loop(
        0, E // WIN // 2, winpair, (jnp.int32(0),) * 4)

    def final(cb, bkt, off, pos):
        k = (16 - (off & 15)) & 15
        plsc.store_compressed(cb.at[pl.ds(off, 16)],
                              jnp.full((16,), PAD_PACK, jnp.int32),
                              mask=iota16 < k)
        off = off + k
        pltpu.sync_copy(cb.at[pl.ds(0, FLUSH)],
                        bins_ref.at[pl.ds(pl.multiple_of(bkt * ECAP + pos, 16), FLUSH)])
        total = pos + off
        pltpu.sync_copy(padb, bins_ref.at[pl.ds(pl.multiple_of(bkt * ECAP + total, 16), FLUSH)])
        stage[pl.ds(0, 16)] = jnp.zeros((16,), jnp.int32) + total
        pltpu.sync_copy(stage, counts_ref.at[pl.ds(pl.multiple_of(bkt * 16, 16), 16)])

    final(cb0, b0, off0, pos0)
    final(cb1, b0 + 1, off1, pos1)


@functools.partial(
    pl.kernel,
    out_type=jax.ShapeDtypeStruct((N * 16,), jnp.float32),
    mesh=_mesh,
    compiler_params=pltpu.CompilerParams(needs_layout_passes=False),
    scratch_types=[
        pltpu.VMEM((ROWS * 16,), jnp.float32),
        pltpu.VMEM((CH,), jnp.int32),
        pltpu.VMEM((16,), jnp.int32),
    ],
)
def _deg_kernel(bins_ref, counts_ref, deg_ref, dega, packv, cntv):
    w = _wid()
    zeros16f = jnp.zeros((16,), jnp.float32)
    onehot = jnp.where(lax.iota(jnp.int32, 16) == 0, 1.0, 0.0)

    for sb in range(2):
        b = 2 * w + sb
        rs0 = b * BS

        def zrow(r, _):
            dega[pl.ds(r * 16, 16)] = zeros16f
            return 0
        lax.fori_loop(0, ROWS, zrow, 0)

        pltpu.sync_copy(counts_ref.at[pl.ds(pl.multiple_of(b * 16, 16), 16)],
                        cntv)
        n = jnp.max(cntv[...])
        nch = (n + CH - 1) >> 7

        def chunk(c, _):
            pos = c * CH
            pltpu.sync_copy(
                bins_ref.at[pl.ds(pl.multiple_of(b * ECAP + pos, 16), CH)],
                packv)

            def grp(g, _):
                loc16 = packv[pl.ds(g * 16, 16)] >> 16
                for j in range(16):
                    loc = loc16[j]
                    plsc.addupdate(dega.at[pl.ds(loc * 16, 16)], onehot)
                return 0
            lax.fori_loop(0, CH // 16, grp, 0)
            return 0

        lax.fori_loop(0, nch, chunk, 0)

        pltpu.sync_copy(dega.at[pl.ds(0, 608 * 16)],
                        deg_ref.at[pl.ds(pl.multiple_of(rs0 * 16, 16),
                                         608 * 16)])

        @pl.when(b < NB - 1)
        def _():
            pltpu.sync_copy(
                dega.at[pl.ds(608 * 16, 176 * 16)],
                deg_ref.at[pl.ds(pl.multiple_of((rs0 + 608) * 16, 16),
                                 176 * 16)])


@functools.partial(
    pl.kernel,
    out_type=jax.ShapeDtypeStruct((N * FP,), jnp.float32),
    mesh=_mesh,
    compiler_params=pltpu.CompilerParams(needs_layout_passes=False),
    scratch_types=[
        pltpu.VMEM((ROWS * FP,), jnp.float32),
        pltpu.VMEM((ANB * ACH,), jnp.int32),
        [pltpu.VMEM((ACH,), jnp.int32) for _ in range(ANB)],
        [pltpu.VMEM((ACH,), jnp.int32) for _ in range(ANB)],
        [pltpu.VMEM((ACH, FP), jnp.float32) for _ in range(ANB)],
        pltpu.VMEM((16,), jnp.int32),
        [pltpu.SemaphoreType.DMA for _ in range(ANB)],
        pltpu.SemaphoreType.DMA,
    ],
)
def _acc_kernel(p_ref, bins_ref, counts_ref, out_ref,
                acc, packq, srcs, locs, rows, cntv, semr, semp):
    w = _wid()
    zeros16f = jnp.zeros((16,), jnp.float32)

    for sb in range(2):
        b = 2 * w + sb
        rs0 = b * BS
        base = b * ECAP

        def zrow(r, _):
            acc[pl.ds(r * 16, 16)] = zeros16f
            return 0
        lax.fori_loop(0, ROWS * FP // 16, zrow, 0)

        pltpu.sync_copy(counts_ref.at[pl.ds(pl.multiple_of(b * 16, 16), 16)], cntv)
        n = jnp.max(cntv[...])
        nq = (n + ANB * ACH - 1) >> 7

        def build_issue(qi, bf):
            for g in range(ACH // 16):
                pk = packq[pl.ds(bf * ACH + g * 16, 16)]
                srcs[bf][pl.ds(g * 16, 16)] = pk & 0xFFFF
                locs[bf][pl.ds(g * 16, 16)] = pk >> 16
            pltpu.async_copy(p_ref.at[srcs[bf]], rows[bf], semr[bf])

        def load_packq(qi):
            pltpu.async_copy(
                bins_ref.at[pl.ds(pl.multiple_of(base + qi * (ANB * ACH), 16),
                                  ANB * ACH)],
                packq, semp)

        iota16 = lax.iota(jnp.int32, 16)

        def accum(bf):
            pltpu.make_async_copy(p_ref.at[srcs[bf]], rows[bf],
                                  semr[bf]).wait()

            def grp(g, _):
                loc16 = locs[bf][pl.ds(g * 16, 16)]
                for j in range(16):
                    loc = loc16[j]
                    e = g * 16 + j
                    base = loc * FP
                    vs = [rows[bf][e, pl.ds(f * 16, 16)]
                          for f in range(FP // 16)]
                    for f in range(FP // 16):
                        plsc.addupdate(acc.at[pl.ds(base + f * 16, 16)],
                                       vs[f])
                return 0
            lax.fori_loop(0, ACH // 16, grp, 0)

        @pl.when(n > 0)
        def _():
            pltpu.async_copy(
                bins_ref.at[pl.ds(pl.multiple_of(base, 16), ANB * ACH)],
                packq, semp)
            pltpu.make_async_copy(
                bins_ref.at[pl.ds(pl.multiple_of(base, 16), ANB * ACH)],
                packq, semp).wait()
            for bf in range(ANB):
                build_issue(0, bf)
            load_packq(1)

            def quad(qi, _):
                pltpu.make_async_copy(
                    bins_ref.at[pl.ds(pl.multiple_of(base, 16), ANB * ACH)],
                    packq, semp).wait()
                for bf in range(ANB):
                    accum(bf)
                    build_issue(qi, bf)
                load_packq(qi + 1)
                return 0
            lax.fori_loop(1, nq, quad, 0)

            pltpu.make_async_copy(
                bins_ref.at[pl.ds(pl.multiple_of(base, 16), ANB * ACH)],
                packq, semp).wait()
            for bf in range(ANB):
                accum(bf)

        pltpu.sync_copy(acc.at[pl.ds(0, 608 * FP)],
                        out_ref.at[pl.ds(pl.multiple_of(rs0 * FP, 16),
                                         608 * FP)])

        @pl.when(b < NB - 1)
        def _():
            pltpu.sync_copy(
                acc.at[pl.ds(608 * FP, 176 * FP)],
                out_ref.at[pl.ds(pl.multiple_of((rs0 + 608) * FP, 16),
                                 176 * FP)])


BK = 512
NBK = 98


def _h0p1_body(x_ref, w0_ref, b0_ref, w1_ref, deg_ref, o_ref):
    h0 = jnp.maximum(
        jnp.dot(x_ref[...], w0_ref[...], preferred_element_type=jnp.float32)
        + b0_ref[...], 0.0)
    dv = lax.rsqrt(deg_ref[:, 0:1] + 1.0)
    o_ref[...] = jnp.dot(h0, w1_ref[...],
                         preferred_element_type=jnp.float32) * dv


def _step_body(agg_ref, p_ref, deg_ref, b_ref, w_ref, o_ref):
    dv = lax.rsqrt(deg_ref[:, 0:1] + 1.0)
    t = (agg_ref[...] + p_ref[...]) * dv
    h = jnp.maximum(t[:, :F] + b_ref[...], 0.0)
    o_ref[...] = jnp.dot(h, w_ref[...],
                         preferred_element_type=jnp.float32) * dv


def _final_body(agg_ref, p_ref, deg_ref, b_ref, batch_ref, g_ref):
    pid = pl.program_id(0)

    @pl.when(pid == 0)
    def _():
        g_ref[...] = jnp.full((G, F), -jnp.inf, jnp.float32)

    dv = lax.rsqrt(deg_ref[:, 0:1] + 1.0)
    t = (agg_ref[...] + p_ref[...]) * dv
    h = jnp.maximum(t[:, :F] + b_ref[...], 0.0)
    rows = pid * BK + lax.broadcasted_iota(jnp.int32, (BK, 1), 0)
    h = jnp.where(rows < N, h, -jnp.inf)
    bt = batch_ref[:, 0:1]
    lo = jnp.min(bt)
    hi = jnp.max(bt)

    def body(gid, _):
        m = bt == gid
        colmax = jnp.max(jnp.where(m, h, -jnp.inf), axis=0)
        cur = g_ref[pl.ds(gid, 1), :]
        g_ref[pl.ds(gid, 1), :] = jnp.maximum(cur, colmax[None, :])
        return 0

    lax.fori_loop(lo, hi + 1, body, 0)


def _head_body(g_ref, w4_ref, b4_ref, w5_ref, b5_ref, w6_ref, b6_ref, o_ref):
    a = jnp.maximum(jnp.dot(g_ref[...], w4_ref[...],
                            preferred_element_type=jnp.float32)
                    + b4_ref[...], 0.0)
    a = jnp.maximum(jnp.dot(a, w5_ref[...],
                            preferred_element_type=jnp.float32)
                    + b5_ref[...], 0.0)
    z = jnp.dot(a, w6_ref[...], preferred_element_type=jnp.float32) \
        + b6_ref[...]
    z = z - jnp.max(z, axis=0, keepdims=True)
    ez = jnp.exp(z)
    o_ref[...] = ez / jnp.sum(ez, axis=0, keepdims=True)


def _rows_spec(width):
    return pl.BlockSpec((BK, width), lambda i: (i, 0))


def _full_spec(shape):
    return pl.BlockSpec(shape, lambda *a: tuple(0 for _ in shape))


def _tc_h0p1(x, w0, b0r, w1p, deg16):
    return pl.pallas_call(
        _h0p1_body,
        grid=(NBK,),
        in_specs=[_rows_spec(19), _full_spec((19, F)), _full_spec((1, F)),
                  _full_spec((F, FP)), _rows_spec(16)],
        out_specs=_rows_spec(FP),
        out_shape=jax.ShapeDtypeStruct((N, FP), jnp.float32),
    )(x, w0, b0r, w1p, deg16)


def _tc_step(agg, p, deg16, br, wp):
    return pl.pallas_call(
        _step_body,
        grid=(NBK,),
        in_specs=[_rows_spec(FP), _rows_spec(FP), _rows_spec(16),
                  _full_spec((1, F)), _full_spec((F, FP))],
        out_specs=_rows_spec(FP),
        out_shape=jax.ShapeDtypeStruct((N, FP), jnp.float32),
    )(agg, p, deg16, br, wp)


def _tc_final(agg, p, deg16, br, batch16):
    return pl.pallas_call(
        _final_body,
        grid=(NBK,),
        in_specs=[_rows_spec(FP), _rows_spec(FP), _rows_spec(16),
                  _full_spec((1, F)), _rows_spec(16)],
        out_specs=_full_spec((G, F)),
        out_shape=jax.ShapeDtypeStruct((G, F), jnp.float32),
    )(agg, p, deg16, br, batch16)


def _tc_head(g, w4, b4r, w5, b5r, w6, b6r):
    return pl.pallas_call(
        _head_body,
        in_specs=[_full_spec((G, F)), _full_spec((F, 80)),
                  _full_spec((1, 80)), _full_spec((80, 60)),
                  _full_spec((1, 60)), _full_spec((60, 6)),
                  _full_spec((1, 6))],
        out_specs=_full_spec((G, 6)),
        out_shape=jax.ShapeDtypeStruct((G, 6), jnp.float32),
    )(g, w4, b4r, w5, b5r, w6, b6r)


def kernel(x, edge_index, batch, W0, b0, W1, b1, W2, b2, W3, b3,
           W4, b4, W5, b5, W6, b6):
    padW = lambda w: jnp.pad(w, ((0, 0), (0, FP - F)))
    w1p, w2p, w3p = padW(W1), padW(W2), padW(W3)
    b0r, b1r, b2r, b3r = (b.reshape(1, F) for b in (b0, b1, b2, b3))
    b4r, b5r, b6r = b4.reshape(1, 80), b5.reshape(1, 60), b6.reshape(1, 6)
    batchp = jnp.concatenate(
        [batch, jnp.full((NBK * BK - N,), G - 1, jnp.int32)])
    batch16 = jnp.broadcast_to(batchp[:, None], (NBK * BK, 16))

    bins, counts = _bin_kernel(edge_index[0], edge_index[1])
    degflat = _deg_kernel(bins, counts)
    deg16 = degflat.reshape(N, 16)

    p1 = _tc_h0p1(x, W0, b0r, w1p, deg16)
    agg1 = _acc_kernel(p1, bins, counts).reshape(N, FP)
    p2 = _tc_step(agg1, p1, deg16, b1r, w2p)
    agg2 = _acc_kernel(p2, bins, counts).reshape(N, FP)
    p3 = _tc_step(agg2, p2, deg16, b2r, w3p)
    agg3 = _acc_kernel(p3, bins, counts).reshape(N, FP)
    g = _tc_final(agg3, p3, deg16, b3r, batch16)
    return _tc_head(g, W4, b4r, W5, b5r, W6, b6r)


# speedup vs baseline: 6.5556x; 1.0091x over previous
import functools

import jax
import jax.numpy as jnp
from jax import lax
from jax.experimental import pallas as pl
from jax.experimental.pallas import tpu as pltpu
from jax.experimental.pallas import tpu_sc as plsc

N = 50000
E = 1600000
G = 64
F = 100
FP = 128
NC, NS = 2, 16
NW = NC * NS
NB = 64
BS = 784
ROWS = 785
DUMP = 784
WIN = 2000
CBCAP = 4096
FLUSH = 2048
CH = 128
ACH = 32
ANB = 4
ECAP = E + FLUSH + 64
PAD_PACK = DUMP << 16
DEGR = 1568

_mesh = plsc.VectorSubcoreMesh(
    core_axis_name="c", subcore_axis_name="s", num_cores=NC, num_subcores=NS)


def _wid():
    return lax.axis_index("s") * NC + lax.axis_index("c")


def _bucket64(d16):
    return ((d16 >> 4) * 5350) >> 18


@functools.partial(
    pl.kernel,
    out_type=(
        jax.ShapeDtypeStruct((NB * ECAP,), jnp.int32),
        jax.ShapeDtypeStruct((NB * 16,), jnp.int32),
    ),
    mesh=_mesh,
    compiler_params=pltpu.CompilerParams(needs_layout_passes=False),
    scratch_types=[
        [pltpu.VMEM((WIN,), jnp.int32) for _ in range(2)],
        [pltpu.VMEM((WIN,), jnp.int32) for _ in range(2)],
        pltpu.VMEM((CBCAP,), jnp.int32),
        pltpu.VMEM((CBCAP,), jnp.int32),
        pltpu.VMEM((FLUSH,), jnp.int32),
        pltpu.VMEM((16,), jnp.int32),
        [pltpu.SemaphoreType.DMA for _ in range(2)],
    ],
)
def _bin_kernel(src_ref, dst_ref, bins_ref, counts_ref,
                wsrc, wdst, cb0, cb1, padb, stage, semw):
    w = _wid()
    b0 = 2 * w
    rs0 = b0 * BS
    iota16 = lax.iota(jnp.int32, 16)

    def initpad(j, _):
        padb[pl.ds(j * 16, 16)] = jnp.full((16,), PAD_PACK, jnp.int32)
        return 0
    lax.fori_loop(0, FLUSH // 16, initpad, 0)

    def issue_win(win, bf):
        base = pl.multiple_of(win * WIN, 16)
        pltpu.async_copy(src_ref.at[pl.ds(base, WIN)], wsrc[bf], semw[bf])
        pltpu.async_copy(dst_ref.at[pl.ds(base, WIN)], wdst[bf], semw[bf])

    def wait_win(win, bf):
        base = pl.multiple_of(win * WIN, 16)
        pltpu.make_async_copy(src_ref.at[pl.ds(base, WIN)], wsrc[bf],
                              semw[bf]).wait()
        pltpu.make_async_copy(dst_ref.at[pl.ds(base, WIN)], wdst[bf],
                              semw[bf]).wait()

    def window(win, bf, carry):
        off0, pos0, off1, pos1 = carry
        wait_win(win, bf)

        def group(g, carry2):
            o0, o1 = carry2
            s16 = wsrc[bf][pl.ds(g * 16, 16)]
            d16 = wdst[bf][pl.ds(g * 16, 16)]
            bk = _bucket64(d16)
            loc = d16 - bk * BS
            pack = s16 | (loc << 16)
            m0 = bk == b0
            m1 = bk == (b0 + 1)
            plsc.store_compressed(cb0.at[pl.ds(o0, 16)], pack, mask=m0)
            plsc.store_compressed(cb1.at[pl.ds(o1, 16)], pack, mask=m1)
            o0 = o0 + plsc.all_reduce_population_count(m0)[0]
            o1 = o1 + plsc.all_reduce_population_count(m1)[0]
            return o0, o1

        off0, off1 = lax.fori_loop(0, WIN // 16, group, (off0, off1))

        @pl.when(win + 2 < E // WIN)
        def _():
            issue_win(win + 2, bf)

        def flush(cb, bkt, off, pos):
            @pl.when(off >= FLUSH)
            def _():
                pltpu.sync_copy(cb.at[pl.ds(0, FLUSH)],
                                bins_ref.at[pl.ds(pl.multiple_of(bkt * ECAP + pos, 16), FLUSH)])

                def shift(j, _):
                    v = cb[pl.ds(FLUSH + j * 16, 16)]
                    cb[pl.ds(j * 16, 16)] = v
                    return 0
                lax.fori_loop(0, (CBCAP - FLUSH) // 16, shift, 0)
            did = (off >= FLUSH).astype(jnp.int32)
            return off - did * FLUSH, pos + did * FLUSH

        off0, pos0 = flush(cb0, b0, off0, pos0)
        off1, pos1 = flush(cb1, b0 + 1, off1, pos1)
        return off0, pos0, off1, pos1

    issue_win(0, 0)
    issue_win(1, 1)

    def winpair(pi, carry):
        carry = window(2 * pi, 0, carry)
        carry = window(2 * pi + 1, 1, carry)
        return carry

    off0, pos0, off1, pos1 = lax.fori_loop(
        0, E // WIN // 2, winpair, (jnp.int32(0),) * 4)

    def final(cb, bkt, off, pos):
        k = (16 - (off & 15)) & 15
        plsc.store_compressed(cb.at[pl.ds(off, 16)],
                              jnp.full((16,), PAD_PACK, jnp.int32),
                              mask=iota16 < k)
        off = off + k
        pltpu.sync_copy(cb.at[pl.ds(0, FLUSH)],
                        bins_ref.at[pl.ds(pl.multiple_of(bkt * ECAP + pos, 16), FLUSH)])
        total = pos + off
        pltpu.sync_copy(padb, bins_ref.at[pl.ds(pl.multiple_of(bkt * ECAP + total, 16), FLUSH)])
        stage[pl.ds(0, 16)] = jnp.zeros((16,), jnp.int32) + total
        pltpu.sync_copy(stage, counts_ref.at[pl.ds(pl.multiple_of(bkt * 16, 16), 16)])

    final(cb0, b0, off0, pos0)
    final(cb1, b0 + 1, off1, pos1)


@functools.partial(
    pl.kernel,
    out_type=jax.ShapeDtypeStruct((N * 16,), jnp.float32),
    mesh=_mesh,
    compiler_params=pltpu.CompilerParams(needs_layout_passes=False),
    scratch_types=[
        pltpu.VMEM((ROWS * 16,), jnp.float32),
        pltpu.VMEM((CH,), jnp.int32),
        pltpu.VMEM((16,), jnp.int32),
    ],
)
def _deg_kernel(bins_ref, counts_ref, deg_ref, dega, packv, cntv):
    w = _wid()
    zeros16f = jnp.zeros((16,), jnp.float32)
    onehot = jnp.where(lax.iota(jnp.int32, 16) == 0, 1.0, 0.0)

    for sb in range(2):
        b = 2 * w + sb
        rs0 = b * BS

        def zrow(r, _):
            dega[pl.ds(r * 16, 16)] = zeros16f
            return 0
        lax.fori_loop(0, ROWS, zrow, 0)

        pltpu.sync_copy(counts_ref.at[pl.ds(pl.multiple_of(b * 16, 16), 16)],
                        cntv)
        n = jnp.max(cntv[...])
        nch = (n + CH - 1) >> 7

        def chunk(c, _):
            pos = c * CH
            pltpu.sync_copy(
                bins_ref.at[pl.ds(pl.multiple_of(b * ECAP + pos, 16), CH)],
                packv)

            def grp(g, _):
                loc16 = packv[pl.ds(g * 16, 16)] >> 16
                loc = loc16[0]
                for j in range(16):
                    loc_next = loc16[j + 1] if j < 15 else loc
                    plsc.addupdate(dega.at[pl.ds(loc * 16, 16)], onehot)
                    loc = loc_next
                return 0
            lax.fori_loop(0, CH // 16, grp, 0)
            return 0

        lax.fori_loop(0, nch, chunk, 0)

        pltpu.sync_copy(dega.at[pl.ds(0, 608 * 16)],
                        deg_ref.at[pl.ds(pl.multiple_of(rs0 * 16, 16),
                                         608 * 16)])

        @pl.when(b < NB - 1)
        def _():
            pltpu.sync_copy(
                dega.at[pl.ds(608 * 16, 176 * 16)],
                deg_ref.at[pl.ds(pl.multiple_of((rs0 + 608) * 16, 16),
                                 176 * 16)])


@functools.partial(
    pl.kernel,
    out_type=jax.ShapeDtypeStruct((N * FP,), jnp.float32),
    mesh=_mesh,
    compiler_params=pltpu.CompilerParams(needs_layout_passes=False),
    scratch_types=[
        pltpu.VMEM((ROWS * FP,), jnp.float32),
        pltpu.VMEM((ANB * ACH,), jnp.int32),
        [pltpu.VMEM((ACH,), jnp.int32) for _ in range(ANB)],
        [pltpu.VMEM((ACH,), jnp.int32) for _ in range(ANB)],
        [pltpu.VMEM((ACH, FP), jnp.float32) for _ in range(ANB)],
        pltpu.VMEM((16,), jnp.int32),
        [pltpu.SemaphoreType.DMA for _ in range(ANB)],
        pltpu.SemaphoreType.DMA,
    ],
)
def _acc_kernel(p_ref, bins_ref, counts_ref, out_ref,
                acc, packq, srcs, locs, rows, cntv, semr, semp):
    w = _wid()
    zeros16f = jnp.zeros((16,), jnp.float32)

    for sb in range(2):
        b = 2 * w + sb
        rs0 = b * BS
        base = b * ECAP

        def zrow(r, _):
            acc[pl.ds(r * 16, 16)] = zeros16f
            return 0
        lax.fori_loop(0, ROWS * FP // 16, zrow, 0)

        pltpu.sync_copy(counts_ref.at[pl.ds(pl.multiple_of(b * 16, 16), 16)], cntv)
        n = jnp.max(cntv[...])
        nq = (n + ANB * ACH - 1) >> 7

        def build_issue(qi, bf):
            for g in range(ACH // 16):
                pk = packq[pl.ds(bf * ACH + g * 16, 16)]
                srcs[bf][pl.ds(g * 16, 16)] = pk & 0xFFFF
                locs[bf][pl.ds(g * 16, 16)] = pk >> 16
            pltpu.async_copy(p_ref.at[srcs[bf]], rows[bf], semr[bf])

        def load_packq(qi):
            pltpu.async_copy(
                bins_ref.at[pl.ds(pl.multiple_of(base + qi * (ANB * ACH), 16),
                                  ANB * ACH)],
                packq, semp)

        iota16 = lax.iota(jnp.int32, 16)

        def accum(bf):
            pltpu.make_async_copy(p_ref.at[srcs[bf]], rows[bf],
                                  semr[bf]).wait()

            def grp(g, _):
                loc16 = locs[bf][pl.ds(g * 16, 16)]

                def loads(j):
                    e = g * 16 + j
                    return [rows[bf][e, pl.ds(f * 16, 16)]
                            for f in range(FP // 16)]

                vs = loads(0)
                loc = loc16[0]
                for j in range(16):
                    if j < 15:
                        vs_next = loads(j + 1)
                        loc_next = loc16[j + 1]
                    base = loc * FP
                    for f in range(FP // 16):
                        plsc.addupdate(acc.at[pl.ds(base + f * 16, 16)],
                                       vs[f])
                    if j < 15:
                        vs, loc = vs_next, loc_next
                return 0
            lax.fori_loop(0, ACH // 16, grp, 0)

        @pl.when(n > 0)
        def _():
            pltpu.async_copy(
                bins_ref.at[pl.ds(pl.multiple_of(base, 16), ANB * ACH)],
                packq, semp)
            pltpu.make_async_copy(
                bins_ref.at[pl.ds(pl.multiple_of(base, 16), ANB * ACH)],
                packq, semp).wait()
            for bf in range(ANB):
                build_issue(0, bf)
            load_packq(1)

            def quad(qi, _):
                pltpu.make_async_copy(
                    bins_ref.at[pl.ds(pl.multiple_of(base, 16), ANB * ACH)],
                    packq, semp).wait()
                for bf in range(ANB):
                    accum(bf)
                    build_issue(qi, bf)
                load_packq(qi + 1)
                return 0
            lax.fori_loop(1, nq, quad, 0)

            pltpu.make_async_copy(
                bins_ref.at[pl.ds(pl.multiple_of(base, 16), ANB * ACH)],
                packq, semp).wait()
            for bf in range(ANB):
                accum(bf)

        pltpu.sync_copy(acc.at[pl.ds(0, 608 * FP)],
                        out_ref.at[pl.ds(pl.multiple_of(rs0 * FP, 16),
                                         608 * FP)])

        @pl.when(b < NB - 1)
        def _():
            pltpu.sync_copy(
                acc.at[pl.ds(608 * FP, 176 * FP)],
                out_ref.at[pl.ds(pl.multiple_of((rs0 + 608) * FP, 16),
                                 176 * FP)])


BK = 512
NBK = 98


def _h0p1_body(x_ref, w0_ref, b0_ref, w1_ref, deg_ref, o_ref):
    h0 = jnp.maximum(
        jnp.dot(x_ref[...], w0_ref[...], preferred_element_type=jnp.float32)
        + b0_ref[...], 0.0)
    dv = lax.rsqrt(deg_ref[:, 0:1] + 1.0)
    o_ref[...] = jnp.dot(h0, w1_ref[...],
                         preferred_element_type=jnp.float32) * dv


def _step_body(agg_ref, p_ref, deg_ref, b_ref, w_ref, o_ref):
    dv = lax.rsqrt(deg_ref[:, 0:1] + 1.0)
    t = (agg_ref[...] + p_ref[...]) * dv
    h = jnp.maximum(t[:, :F] + b_ref[...], 0.0)
    o_ref[...] = jnp.dot(h, w_ref[...],
                         preferred_element_type=jnp.float32) * dv


def _final_body(agg_ref, p_ref, deg_ref, b_ref, batch_ref, g_ref):
    pid = pl.program_id(0)

    @pl.when(pid == 0)
    def _():
        g_ref[...] = jnp.full((G, F), -jnp.inf, jnp.float32)

    dv = lax.rsqrt(deg_ref[:, 0:1] + 1.0)
    t = (agg_ref[...] + p_ref[...]) * dv
    h = jnp.maximum(t[:, :F] + b_ref[...], 0.0)
    rows = pid * BK + lax.broadcasted_iota(jnp.int32, (BK, 1), 0)
    h = jnp.where(rows < N, h, -jnp.inf)
    bt = batch_ref[:, 0:1]
    lo = jnp.min(bt)
    hi = jnp.max(bt)

    def body(gid, _):
        m = bt == gid
        colmax = jnp.max(jnp.where(m, h, -jnp.inf), axis=0)
        cur = g_ref[pl.ds(gid, 1), :]
        g_ref[pl.ds(gid, 1), :] = jnp.maximum(cur, colmax[None, :])
        return 0

    lax.fori_loop(lo, hi + 1, body, 0)


def _head_body(g_ref, w4_ref, b4_ref, w5_ref, b5_ref, w6_ref, b6_ref, o_ref):
    a = jnp.maximum(jnp.dot(g_ref[...], w4_ref[...],
                            preferred_element_type=jnp.float32)
                    + b4_ref[...], 0.0)
    a = jnp.maximum(jnp.dot(a, w5_ref[...],
                            preferred_element_type=jnp.float32)
                    + b5_ref[...], 0.0)
    z = jnp.dot(a, w6_ref[...], preferred_element_type=jnp.float32) \
        + b6_ref[...]
    z = z - jnp.max(z, axis=0, keepdims=True)
    ez = jnp.exp(z)
    o_ref[...] = ez / jnp.sum(ez, axis=0, keepdims=True)


def _rows_spec(width):
    return pl.BlockSpec((BK, width), lambda i: (i, 0))


def _full_spec(shape):
    return pl.BlockSpec(shape, lambda *a: tuple(0 for _ in shape))


def _tc_h0p1(x, w0, b0r, w1p, deg16):
    return pl.pallas_call(
        _h0p1_body,
        grid=(NBK,),
        in_specs=[_rows_spec(19), _full_spec((19, F)), _full_spec((1, F)),
                  _full_spec((F, FP)), _rows_spec(16)],
        out_specs=_rows_spec(FP),
        out_shape=jax.ShapeDtypeStruct((N, FP), jnp.float32),
    )(x, w0, b0r, w1p, deg16)


def _tc_step(agg, p, deg16, br, wp):
    return pl.pallas_call(
        _step_body,
        grid=(NBK,),
        in_specs=[_rows_spec(FP), _rows_spec(FP), _rows_spec(16),
                  _full_spec((1, F)), _full_spec((F, FP))],
        out_specs=_rows_spec(FP),
        out_shape=jax.ShapeDtypeStruct((N, FP), jnp.float32),
    )(agg, p, deg16, br, wp)


def _tc_final(agg, p, deg16, br, batch16):
    return pl.pallas_call(
        _final_body,
        grid=(NBK,),
        in_specs=[_rows_spec(FP), _rows_spec(FP), _rows_spec(16),
                  _full_spec((1, F)), _rows_spec(16)],
        out_specs=_full_spec((G, F)),
        out_shape=jax.ShapeDtypeStruct((G, F), jnp.float32),
    )(agg, p, deg16, br, batch16)


def _tc_head(g, w4, b4r, w5, b5r, w6, b6r):
    return pl.pallas_call(
        _head_body,
        in_specs=[_full_spec((G, F)), _full_spec((F, 80)),
                  _full_spec((1, 80)), _full_spec((80, 60)),
                  _full_spec((1, 60)), _full_spec((60, 6)),
                  _full_spec((1, 6))],
        out_specs=_full_spec((G, 6)),
        out_shape=jax.ShapeDtypeStruct((G, 6), jnp.float32),
    )(g, w4, b4r, w5, b5r, w6, b6r)


def kernel(x, edge_index, batch, W0, b0, W1, b1, W2, b2, W3, b3,
           W4, b4, W5, b5, W6, b6):
    padW = lambda w: jnp.pad(w, ((0, 0), (0, FP - F)))
    w1p, w2p, w3p = padW(W1), padW(W2), padW(W3)
    b0r, b1r, b2r, b3r = (b.reshape(1, F) for b in (b0, b1, b2, b3))
    b4r, b5r, b6r = b4.reshape(1, 80), b5.reshape(1, 60), b6.reshape(1, 6)
    batchp = jnp.concatenate(
        [batch, jnp.full((NBK * BK - N,), G - 1, jnp.int32)])
    batch16 = jnp.broadcast_to(batchp[:, None], (NBK * BK, 16))

    bins, counts = _bin_kernel(edge_index[0], edge_index[1])
    degflat = _deg_kernel(bins, counts)
    deg16 = degflat.reshape(N, 16)

    p1 = _tc_h0p1(x, W0, b0r, w1p, deg16)
    agg1 = _acc_kernel(p1, bins, counts).reshape(N, FP)
    p2 = _tc_step(agg1, p1, deg16, b1r, w2p)
    agg2 = _acc_kernel(p2, bins, counts).reshape(N, FP)
    p3 = _tc_step(agg2, p2, deg16, b2r, w3p)
    agg3 = _acc_kernel(p3, bins, counts).reshape(N, FP)
    g = _tc_final(agg3, p3, deg16, b3r, batch16)
    return _tc_head(g, W4, b4r, W5, b5r, W6, b6r)
